# baseline (speedup 1.0000x reference)
import jax
import jax.numpy as jnp
from jax import lax
from jax.experimental import pallas as pl
from jax.experimental.pallas import tpu as pltpu
from jax.experimental.pallas import tpu_sc as plsc

_SEG_DIMS = [5, 6, 2]
_EMB_DIM = 64
_E = 800000

_NC = 2
_NS = 16
_NW = _NC * _NS
_CHUNK = 128
_NCHUNKS = _E // _CHUNK
_GROUPS = _CHUNK // 16
_NBUF = 3
_NROWS = _SEG_DIMS[0] * _SEG_DIMS[1] * _SEG_DIMS[2]


def _seg_argmax(cols):
    best = cols[0]
    bidx = jnp.zeros((16,), jnp.int32)
    for j in range(1, len(cols)):
        m = cols[j] > best
        bidx = jnp.where(m, jnp.full((16,), j, jnp.int32), bidx)
        best = jnp.where(m, cols[j], best)
    return bidx


def _body(ea_hbm, t_hbm, out_hbm, t_v, *scratch):
    ea_v = scratch[0:_NBUF]
    rows_v = scratch[_NBUF:2 * _NBUF]
    in_sem = scratch[2 * _NBUF:3 * _NBUF]
    o_sem = scratch[3 * _NBUF:4 * _NBUF]

    wid = lax.axis_index("s") * _NC + lax.axis_index("c")
    n_my = (_NCHUNKS - wid + _NW - 1) // _NW

    pltpu.sync_copy(t_hbm, t_v)

    def chunk_of(j):
        return wid + j * _NW

    def start_in(b, j):
        base = chunk_of(j) * _CHUNK * 13
        pltpu.make_async_copy(
            ea_hbm.at[pl.ds(base, _CHUNK * 13)], ea_v[b], in_sem[b]
        ).start()

    ramp = lax.iota(jnp.int32, 16)
    ramp13 = ramp * 13
    ramp_q = [ramp + q * 16 for q in range(4)]

    def process(b):
        def grp(g, carry):
            flat = g * (16 * 13) + ramp13
            cols = [
                plsc.load_gather(ea_v[b], [flat + jnp.full((16,), j, jnp.int32)])
                for j in range(13)
            ]
            i0 = _seg_argmax(cols[0:5])
            i1 = _seg_argmax(cols[5:11])
            i2 = _seg_argmax(cols[11:13])
            cbase = (i0 * 12 + i1 * 2 + i2) * _EMB_DIM
            gbase = g * (16 * _EMB_DIM)
            for e in range(16):
                bc = cbase.at[jnp.full((16,), e, jnp.int32)].get(
                    mode="promise_in_bounds"
                )
                for q in range(4):
                    vals = plsc.load_gather(t_v, [bc + ramp_q[q]])
                    rows_v[b][pl.ds(gbase + e * _EMB_DIM + q * 16, 16)] = vals
            return carry

        lax.fori_loop(0, _GROUPS, grp, jnp.int32(0))

    for b in range(_NBUF):
        @pl.when(b < n_my)
        def _(b=b):
            start_in(b, jnp.int32(b))

    n_outer = (_NCHUNKS // _NW + 1 + 2 * _NBUF - 1) // _NBUF

    def outer(o, carry):
        for b in range(_NBUF):
            j = o * _NBUF + b

            @pl.when(jnp.logical_and(j >= _NBUF, j - _NBUF < n_my))
            def _():
                pltpu.make_async_copy(
                    rows_v[b],
                    out_hbm.at[pl.ds(0, _CHUNK * _EMB_DIM)],
                    o_sem[b],
                ).wait()

            @pl.when(j < n_my)
            def _():
                pltpu.make_async_copy(
                    ea_hbm.at[pl.ds(0, _CHUNK * 13)], ea_v[b], in_sem[b]
                ).wait()
                process(b)
                pltpu.make_async_copy(
                    rows_v[b],
                    out_hbm.at[
                        pl.ds(chunk_of(j) * _CHUNK * _EMB_DIM, _CHUNK * _EMB_DIM)
                    ],
                    o_sem[b],
                ).start()

                @pl.when(j + _NBUF < n_my)
                def _():
                    start_in(b, j + _NBUF)

        return carry

    lax.fori_loop(0, n_outer, outer, jnp.int32(0), unroll=False)


@jax.jit
def kernel(edge_attr, W0, W1, W2):
    table = (
        W0[:, None, None, :] + W1[None, :, None, :] + W2[None, None, :, :]
    ).reshape(_NROWS * _EMB_DIM)

    scratch = (
        [pltpu.VMEM((_CHUNK * 13,), jnp.float32) for _ in range(_NBUF)]
        + [pltpu.VMEM((_CHUNK * _EMB_DIM,), jnp.float32) for _ in range(_NBUF)]
        + [pltpu.SemaphoreType.DMA for _ in range(2 * _NBUF)]
    )
    run = pl.kernel(
        _body,
        out_type=jax.ShapeDtypeStruct((_E * _EMB_DIM,), jnp.float32),
        mesh=plsc.VectorSubcoreMesh(core_axis_name="c", subcore_axis_name="s"),
        scratch_types=[pltpu.VMEM((_NROWS * _EMB_DIM,), jnp.float32)] + scratch,
        compiler_params=pltpu.CompilerParams(
            needs_layout_passes=False, use_tc_tiling_on_sc=False
        ),
    )
    out = run(edge_attr.reshape(-1), table)
    return out.reshape(_E, _EMB_DIM)


# speedup vs baseline: 1.2449x; 1.2449x over previous
import jax
import jax.numpy as jnp
from jax import lax
from jax.experimental import pallas as pl
from jax.experimental.pallas import tpu as pltpu
from jax.experimental.pallas import tpu_sc as plsc

_SEG_DIMS = [5, 6, 2]
_EMB_DIM = 64
_E = 800000

_NC = 2
_NS = 16
_NW = _NC * _NS
_CHUNK = 128
_NCHUNKS = _E // _CHUNK
_GROUPS = _CHUNK // 16
_NBUF = 3
_NROWS = _SEG_DIMS[0] * _SEG_DIMS[1] * _SEG_DIMS[2]


def _seg_argmax(cols):
    best = cols[0]
    bidx = jnp.zeros((16,), jnp.int32)
    for j in range(1, len(cols)):
        m = cols[j] > best
        bidx = jnp.where(m, jnp.full((16,), j, jnp.int32), bidx)
        best = jnp.where(m, cols[j], best)
    return bidx


def _body(ea_hbm, t_hbm, out_hbm, t_v, *scratch):
    ea_v = scratch[0:_NBUF]
    rows_v = scratch[_NBUF:2 * _NBUF]
    in_sem = scratch[2 * _NBUF:3 * _NBUF]
    o_sem = scratch[3 * _NBUF:4 * _NBUF]

    wid = lax.axis_index("s") * _NC + lax.axis_index("c")
    n_my = (_NCHUNKS - wid + _NW - 1) // _NW

    pltpu.sync_copy(t_hbm, t_v)

    def chunk_of(j):
        return wid + j * _NW

    def start_in(b, j):
        base = chunk_of(j) * _CHUNK
        pltpu.make_async_copy(
            ea_hbm.at[pl.ds(base, _CHUNK)], ea_v[b], in_sem[b]
        ).start()

    ramp = lax.iota(jnp.int32, 16)
    ramp_q = [ramp + q * 16 for q in range(4)]

    def process(b):
        def grp(g, carry):
            rows16 = g * 16 + ramp
            cols = [
                plsc.load_gather(
                    ea_v[b], [rows16, jnp.full((16,), j, jnp.int32)]
                )
                for j in range(13)
            ]
            i0 = _seg_argmax(cols[0:5])
            i1 = _seg_argmax(cols[5:11])
            i2 = _seg_argmax(cols[11:13])
            cidx = i0 * 12 + i1 * 2 + i2
            for e in range(16):
                bc = cidx.at[jnp.full((16,), e, jnp.int32)].get(
                    mode="promise_in_bounds"
                )
                erow = g * 16 + e
                for q in range(4):
                    vals = plsc.load_gather(t_v, [bc, ramp_q[q]])
                    rows_v[b][erow, pl.ds(q * 16, 16)] = vals
            return carry

        lax.fori_loop(0, _GROUPS, grp, jnp.int32(0))

    for b in range(_NBUF):
        @pl.when(b < n_my)
        def _(b=b):
            start_in(b, jnp.int32(b))

    n_outer = (_NCHUNKS // _NW + 1 + 2 * _NBUF - 1) // _NBUF

    def outer(o, carry):
        for b in range(_NBUF):
            j = o * _NBUF + b

            @pl.when(jnp.logical_and(j >= _NBUF, j - _NBUF < n_my))
            def _():
                pltpu.make_async_copy(
                    rows_v[b],
                    out_hbm.at[pl.ds(0, _CHUNK)],
                    o_sem[b],
                ).wait()

            @pl.when(j < n_my)
            def _():
                pltpu.make_async_copy(
                    ea_hbm.at[pl.ds(0, _CHUNK)], ea_v[b], in_sem[b]
                ).wait()
                process(b)
                pltpu.make_async_copy(
                    rows_v[b],
                    out_hbm.at[pl.ds(chunk_of(j) * _CHUNK, _CHUNK)],
                    o_sem[b],
                ).start()

                @pl.when(j + _NBUF < n_my)
                def _():
                    start_in(b, j + _NBUF)

        return carry

    lax.fori_loop(0, n_outer, outer, jnp.int32(0), unroll=False)


@jax.jit
def kernel(edge_attr, W0, W1, W2):
    table = (
        W0[:, None, None, :] + W1[None, :, None, :] + W2[None, None, :, :]
    ).reshape(_NROWS, _EMB_DIM)

    scratch = (
        [pltpu.VMEM((_CHUNK, 13), jnp.float32) for _ in range(_NBUF)]
        + [pltpu.VMEM((_CHUNK, _EMB_DIM), jnp.float32) for _ in range(_NBUF)]
        + [pltpu.SemaphoreType.DMA for _ in range(2 * _NBUF)]
    )
    run = pl.kernel(
        _body,
        out_type=jax.ShapeDtypeStruct((_E, _EMB_DIM), jnp.float32),
        mesh=plsc.VectorSubcoreMesh(core_axis_name="c", subcore_axis_name="s"),
        scratch_types=[pltpu.VMEM((_NROWS, _EMB_DIM), jnp.float32)] + scratch,
        compiler_params=pltpu.CompilerParams(
            needs_layout_passes=False, use_tc_tiling_on_sc=True
        ),
    )
    return run(edge_attr, table)


# speedup vs baseline: 2.5730x; 2.0668x over previous
import jax
import jax.numpy as jnp
from jax import lax
from jax.experimental import pallas as pl
from jax.experimental.pallas import tpu as pltpu
from jax.experimental.pallas import tpu_sc as plsc

_SEG_DIMS = [5, 6, 2]
_EMB_DIM = 64
_E = 800000

_NC = 2
_NS = 16
_NW = _NC * _NS
_CHUNK = 128
_NCHUNKS = _E // _CHUNK
_GROUPS = _CHUNK // 16
_NBUF = 3
_NROWS = _SEG_DIMS[0] * _SEG_DIMS[1] * _SEG_DIMS[2]
_TSTRIDE = _EMB_DIM + 1


def _seg_argmax(cols):
    best = cols[0]
    bidx = jnp.zeros((16,), jnp.int32)
    for j in range(1, len(cols)):
        m = cols[j] > best
        bidx = jnp.where(m, jnp.full((16,), j, jnp.int32), bidx)
        best = jnp.where(m, cols[j], best)
    return bidx


def _body(ea_hbm, t_hbm, out_hbm, t_tmp, t65, *scratch):
    ea_v = scratch[0:_NBUF]
    rows_v = scratch[_NBUF:2 * _NBUF]
    in_sem = scratch[2 * _NBUF:3 * _NBUF]
    o_sem = scratch[3 * _NBUF:4 * _NBUF]

    wid = lax.axis_index("s") * _NC + lax.axis_index("c")
    n_my = (_NCHUNKS - wid + _NW - 1) // _NW

    pltpu.sync_copy(t_hbm, t_tmp)

    def t_row(r, carry):
        for q in range(4):
            t65[pl.ds(r * _TSTRIDE + q * 16, 16)] = t_tmp[r, pl.ds(q * 16, 16)]
        return carry

    lax.fori_loop(0, _NROWS, t_row, jnp.int32(0))

    def chunk_of(j):
        return wid + j * _NW

    def start_in(b, j):
        base = chunk_of(j) * _CHUNK
        pltpu.make_async_copy(
            ea_hbm.at[:, pl.ds(base, _CHUNK)], ea_v[b], in_sem[b]
        ).start()

    def process(b):
        def grp(g, carry):
            cols = [ea_v[b][j, pl.ds(g * 16, 16)] for j in range(13)]
            i0 = _seg_argmax(cols[0:5])
            i1 = _seg_argmax(cols[5:11])
            i2 = _seg_argmax(cols[11:13])
            c65 = (i0 * 12 + i1 * 2 + i2) * _TSTRIDE
            for c in range(_EMB_DIM):
                vals = plsc.load_gather(t65, [c65 + jnp.full((16,), c, jnp.int32)])
                rows_v[b][c, pl.ds(g * 16, 16)] = vals
            return carry

        lax.fori_loop(0, _GROUPS, grp, jnp.int32(0))

    for b in range(_NBUF):
        @pl.when(b < n_my)
        def _(b=b):
            start_in(b, jnp.int32(b))

    n_outer = (_NCHUNKS // _NW + 1 + 2 * _NBUF - 1) // _NBUF

    def outer(o, carry):
        for b in range(_NBUF):
            j = o * _NBUF + b

            @pl.when(jnp.logical_and(j >= _NBUF, j - _NBUF < n_my))
            def _():
                pltpu.make_async_copy(
                    rows_v[b],
                    out_hbm.at[:, pl.ds(0, _CHUNK)],
                    o_sem[b],
                ).wait()

            @pl.when(j < n_my)
            def _():
                pltpu.make_async_copy(
                    ea_hbm.at[:, pl.ds(0, _CHUNK)], ea_v[b], in_sem[b]
                ).wait()
                process(b)
                pltpu.make_async_copy(
                    rows_v[b],
                    out_hbm.at[:, pl.ds(chunk_of(j) * _CHUNK, _CHUNK)],
                    o_sem[b],
                ).start()

                @pl.when(j + _NBUF < n_my)
                def _():
                    start_in(b, j + _NBUF)

        return carry

    lax.fori_loop(0, n_outer, outer, jnp.int32(0), unroll=False)


@jax.jit
def kernel(edge_attr, W0, W1, W2):
    table = (
        W0[:, None, None, :] + W1[None, :, None, :] + W2[None, None, :, :]
    ).reshape(_NROWS, _EMB_DIM)

    scratch = (
        [pltpu.VMEM((13, _CHUNK), jnp.float32) for _ in range(_NBUF)]
        + [pltpu.VMEM((_EMB_DIM, _CHUNK), jnp.float32) for _ in range(_NBUF)]
        + [pltpu.SemaphoreType.DMA for _ in range(2 * _NBUF)]
    )
    run = pl.kernel(
        _body,
        out_type=jax.ShapeDtypeStruct((_EMB_DIM, _E), jnp.float32),
        mesh=plsc.VectorSubcoreMesh(core_axis_name="c", subcore_axis_name="s"),
        scratch_types=[
            pltpu.VMEM((_NROWS, _EMB_DIM), jnp.float32),
            pltpu.VMEM((_NROWS * _TSTRIDE, ), jnp.float32),
        ] + scratch,
        compiler_params=pltpu.CompilerParams(
            needs_layout_passes=False, use_tc_tiling_on_sc=True
        ),
    )
    out_t = run(edge_attr.T, table)
    return out_t.T


# speedup vs baseline: 2.6757x; 1.0399x over previous
import jax
import jax.numpy as jnp
from jax import lax
from jax.experimental import pallas as pl
from jax.experimental.pallas import tpu as pltpu
from jax.experimental.pallas import tpu_sc as plsc

_SEG_DIMS = [5, 6, 2]
_EMB_DIM = 64
_E = 800000

_NC = 2
_NS = 16
_NW = _NC * _NS
_CHUNK = 128
_NCHUNKS = _E // _CHUNK
_GROUPS = _CHUNK // 16
_NBUF = 3
_NROWS = _SEG_DIMS[0] * _SEG_DIMS[1] * _SEG_DIMS[2]
_REP = _NROWS * _EMB_DIM + 1


def _seg_argmax(cols):
    best = cols[0]
    bidx = jnp.zeros((16,), jnp.int32)
    for j in range(1, len(cols)):
        m = cols[j] > best
        bidx = jnp.where(m, jnp.full((16,), j, jnp.int32), bidx)
        best = jnp.where(m, cols[j], best)
    return bidx


def _body(ea_hbm, t_hbm, out_hbm, t_tmp, t65, *scratch):
    ea_v = scratch[0:_NBUF]
    rows_v = scratch[_NBUF:2 * _NBUF]
    in_sem = scratch[2 * _NBUF:3 * _NBUF]
    o_sem = scratch[3 * _NBUF:4 * _NBUF]

    wid = lax.axis_index("s") * _NC + lax.axis_index("c")
    n_my = (_NCHUNKS - wid + _NW - 1) // _NW

    pltpu.sync_copy(t_hbm, t_tmp)

    def t_row(r, carry):
        for q in range(4):
            vals = t_tmp[r, pl.ds(q * 16, 16)]
            for rep in range(16):
                t65[pl.ds(rep * _REP + r * _EMB_DIM + q * 16, 16)] = vals
        return carry

    lax.fori_loop(0, _NROWS, t_row, jnp.int32(0))

    def chunk_of(j):
        return wid + j * _NW

    def start_in(b, j):
        base = chunk_of(j) * _CHUNK
        pltpu.make_async_copy(
            ea_hbm.at[:, pl.ds(base, _CHUNK)], ea_v[b], in_sem[b]
        ).start()

    lane_rep = lax.iota(jnp.int32, 16) * _REP

    def process(b):
        def grp(g, carry):
            cols = [ea_v[b][j, pl.ds(g * 16, 16)] for j in range(13)]
            i0 = _seg_argmax(cols[0:5])
            i1 = _seg_argmax(cols[5:11])
            i2 = _seg_argmax(cols[11:13])
            c65 = (i0 * 12 + i1 * 2 + i2) * _EMB_DIM + lane_rep
            for c in range(_EMB_DIM):
                vals = plsc.load_gather(t65, [c65 + jnp.full((16,), c, jnp.int32)])
                rows_v[b][c, pl.ds(g * 16, 16)] = vals
            return carry

        lax.fori_loop(0, _GROUPS, grp, jnp.int32(0))

    for b in range(_NBUF):
        @pl.when(b < n_my)
        def _(b=b):
            start_in(b, jnp.int32(b))

    n_outer = (_NCHUNKS // _NW + 1 + 2 * _NBUF - 1) // _NBUF

    def outer(o, carry):
        for b in range(_NBUF):
            j = o * _NBUF + b

            @pl.when(jnp.logical_and(j >= _NBUF, j - _NBUF < n_my))
            def _():
                pltpu.make_async_copy(
                    rows_v[b],
                    out_hbm.at[:, pl.ds(0, _CHUNK)],
                    o_sem[b],
                ).wait()

            @pl.when(j < n_my)
            def _():
                pltpu.make_async_copy(
                    ea_hbm.at[:, pl.ds(0, _CHUNK)], ea_v[b], in_sem[b]
                ).wait()
                process(b)
                pltpu.make_async_copy(
                    rows_v[b],
                    out_hbm.at[:, pl.ds(chunk_of(j) * _CHUNK, _CHUNK)],
                    o_sem[b],
                ).start()

                @pl.when(j + _NBUF < n_my)
                def _():
                    start_in(b, j + _NBUF)

        return carry

    lax.fori_loop(0, n_outer, outer, jnp.int32(0), unroll=False)


@jax.jit
def kernel(edge_attr, W0, W1, W2):
    table = (
        W0[:, None, None, :] + W1[None, :, None, :] + W2[None, None, :, :]
    ).reshape(_NROWS, _EMB_DIM)

    scratch = (
        [pltpu.VMEM((13, _CHUNK), jnp.float32) for _ in range(_NBUF)]
        + [pltpu.VMEM((_EMB_DIM, _CHUNK), jnp.float32) for _ in range(_NBUF)]
        + [pltpu.SemaphoreType.DMA for _ in range(2 * _NBUF)]
    )
    run = pl.kernel(
        _body,
        out_type=jax.ShapeDtypeStruct((_EMB_DIM, _E), jnp.float32),
        mesh=plsc.VectorSubcoreMesh(core_axis_name="c", subcore_axis_name="s"),
        scratch_types=[
            pltpu.VMEM((_NROWS, _EMB_DIM), jnp.float32),
            pltpu.VMEM((16 * _REP, ), jnp.float32),
        ] + scratch,
        compiler_params=pltpu.CompilerParams(
            needs_layout_passes=False, use_tc_tiling_on_sc=True
        ),
    )
    out_t = run(edge_attr.T, table)
    return out_t.T


# speedup vs baseline: 2.6808x; 1.0019x over previous
import jax
import jax.numpy as jnp
from jax import lax
from jax.experimental import pallas as pl
from jax.experimental.pallas import tpu as pltpu
from jax.experimental.pallas import tpu_sc as plsc

_SEG_DIMS = [5, 6, 2]
_EMB_DIM = 64
_E = 800000

_NC = 2
_NS = 16
_NW = _NC * _NS
_CHUNK = 128
_NCHUNKS = _E // _CHUNK
_GROUPS = _CHUNK // 16
_NBUF = 3
_NROWS = _SEG_DIMS[0] * _SEG_DIMS[1] * _SEG_DIMS[2]
_REP = _NROWS * _EMB_DIM + 1


def _seg_argmax(cols):
    best = cols[0]
    bidx = jnp.zeros((16,), jnp.int32)
    for j in range(1, len(cols)):
        m = cols[j] > best
        bidx = jnp.where(m, jnp.full((16,), j, jnp.int32), bidx)
        best = jnp.where(m, cols[j], best)
    return bidx


def _body(ea_hbm, t_hbm, out_hbm, t_tmp, t65, *scratch):
    ea_v = scratch[0:_NBUF]
    rows_v = scratch[_NBUF:2 * _NBUF]
    in_sem = scratch[2 * _NBUF:3 * _NBUF]
    o_sem = scratch[3 * _NBUF:4 * _NBUF]

    wid = lax.axis_index("s") * _NC + lax.axis_index("c")
    n_my = (_NCHUNKS - wid + _NW - 1) // _NW

    pltpu.sync_copy(t_hbm, t_tmp)

    def t_row(r, carry):
        for q in range(4):
            vals = t_tmp[r, pl.ds(q * 16, 16)]
            for rep in range(16):
                t65[pl.ds(rep * _REP + r * _EMB_DIM + q * 16, 16)] = vals
        return carry

    lax.fori_loop(0, _NROWS, t_row, jnp.int32(0))

    def chunk_of(j):
        return wid + j * _NW

    def start_in(b, j):
        base = chunk_of(j) * _CHUNK
        pltpu.make_async_copy(
            ea_hbm.at[:, pl.ds(base, _CHUNK)], ea_v[b], in_sem[b]
        ).start()

    lane_rep = lax.iota(jnp.int32, 16) * _REP

    def process(b):
        @plsc.parallel_loop(0, _GROUPS, 1, unroll=2)
        def grp(g):
            cols = [ea_v[b][j, pl.ds(g * 16, 16)] for j in range(13)]
            i0 = _seg_argmax(cols[0:5])
            i1 = _seg_argmax(cols[5:11])
            i2 = _seg_argmax(cols[11:13])
            c65 = (i0 * 12 + i1 * 2 + i2) * _EMB_DIM + lane_rep
            for c0 in range(0, _EMB_DIM, 16):
                vals = [
                    plsc.load_gather(
                        t65, [c65 + jnp.full((16,), c0 + k, jnp.int32)]
                    )
                    for k in range(16)
                ]
                for k in range(16):
                    rows_v[b][c0 + k, pl.ds(g * 16, 16)] = vals[k]

    for b in range(_NBUF):
        @pl.when(b < n_my)
        def _(b=b):
            start_in(b, jnp.int32(b))

    n_outer = (_NCHUNKS // _NW + 1 + 2 * _NBUF - 1) // _NBUF

    def outer(o, carry):
        for b in range(_NBUF):
            j = o * _NBUF + b

            @pl.when(jnp.logical_and(j >= _NBUF, j - _NBUF < n_my))
            def _():
                pltpu.make_async_copy(
                    rows_v[b],
                    out_hbm.at[:, pl.ds(0, _CHUNK)],
                    o_sem[b],
                ).wait()

            @pl.when(j < n_my)
            def _():
                pltpu.make_async_copy(
                    ea_hbm.at[:, pl.ds(0, _CHUNK)], ea_v[b], in_sem[b]
                ).wait()
                process(b)
                pltpu.make_async_copy(
                    rows_v[b],
                    out_hbm.at[:, pl.ds(chunk_of(j) * _CHUNK, _CHUNK)],
                    o_sem[b],
                ).start()

                @pl.when(j + _NBUF < n_my)
                def _():
                    start_in(b, j + _NBUF)

        return carry

    lax.fori_loop(0, n_outer, outer, jnp.int32(0), unroll=False)


@jax.jit
def kernel(edge_attr, W0, W1, W2):
    table = (
        W0[:, None, None, :] + W1[None, :, None, :] + W2[None, None, :, :]
    ).reshape(_NROWS, _EMB_DIM)

    scratch = (
        [pltpu.VMEM((13, _CHUNK), jnp.float32) for _ in range(_NBUF)]
        + [pltpu.VMEM((_EMB_DIM, _CHUNK), jnp.float32) for _ in range(_NBUF)]
        + [pltpu.SemaphoreType.DMA for _ in range(2 * _NBUF)]
    )
    run = pl.kernel(
        _body,
        out_type=jax.ShapeDtypeStruct((_EMB_DIM, _E), jnp.float32),
        mesh=plsc.VectorSubcoreMesh(core_axis_name="c", subcore_axis_name="s"),
        scratch_types=[
            pltpu.VMEM((_NROWS, _EMB_DIM), jnp.float32),
            pltpu.VMEM((16 * _REP, ), jnp.float32),
        ] + scratch,
        compiler_params=pltpu.CompilerParams(
            needs_layout_passes=False, use_tc_tiling_on_sc=True
        ),
    )
    out_t = run(edge_attr.T, table)
    return out_t.T


# speedup vs baseline: 3.6605x; 1.3654x over previous
import jax
import jax.numpy as jnp
from jax import lax
from jax.experimental import pallas as pl
from jax.experimental.pallas import tpu as pltpu
from jax.experimental.pallas import tpu_sc as plsc

_SEG_DIMS = [5, 6, 2]
_EMB_DIM = 64
_E = 800000

_NC = 2
_NS = 16
_NW = _NC * _NS
_CHUNK = 128
_NCHUNKS = _E // _CHUNK
_GROUPS = _CHUNK // 16
_NBUF = 3
_NROWS = _SEG_DIMS[0] * _SEG_DIMS[1] * _SEG_DIMS[2]
_REP = _NROWS * _EMB_DIM + 1


def _seg_argmax(cols):
    best = cols[0]
    bidx = jnp.zeros((16,), jnp.int32)
    for j in range(1, len(cols)):
        m = cols[j] > best
        bidx = jnp.where(m, jnp.full((16,), j, jnp.int32), bidx)
        best = jnp.where(m, cols[j], best)
    return bidx


def _body(ea_hbm, t_hbm, out_hbm, t_tmp, t65, *scratch):
    ea_v = scratch[0:_NBUF]
    rows_v = scratch[_NBUF:2 * _NBUF]
    in_sem = scratch[2 * _NBUF:3 * _NBUF]
    o_sem = scratch[3 * _NBUF:4 * _NBUF]

    wid = lax.axis_index("s") * _NC + lax.axis_index("c")
    n_my = (_NCHUNKS - wid + _NW - 1) // _NW

    pltpu.sync_copy(t_hbm, t_tmp)

    def t_row(r, carry):
        for q in range(4):
            vals = t_tmp[r, pl.ds(q * 16, 16)]
            for rep in range(16):
                t65[pl.ds(rep * _REP + r * _EMB_DIM + q * 16, 16)] = vals
        return carry

    lax.fori_loop(0, _NROWS, t_row, jnp.int32(0))

    def chunk_of(j):
        return wid + j * _NW

    def start_in(b, j):
        base = chunk_of(j) * _CHUNK
        pltpu.make_async_copy(
            ea_hbm.at[:, pl.ds(base, _CHUNK)], ea_v[b], in_sem[b]
        ).start()

    lane_rep = lax.iota(jnp.int32, 16) * _REP

    def process(b):
        @plsc.parallel_loop(0, _GROUPS, 1, unroll=2)
        def grp(g):
            cols = [ea_v[b][j, pl.ds(g * 16, 16)] for j in range(13)]
            i0 = _seg_argmax(cols[0:5])
            i1 = _seg_argmax(cols[5:11])
            i2 = _seg_argmax(cols[11:13])
            c65 = (i0 * 12 + i1 * 2 + i2) * _EMB_DIM + lane_rep
            skew = 4
            vals = {}
            for c in range(_EMB_DIM + skew):
                if c < _EMB_DIM:
                    vals[c] = plsc.load_gather(
                        t65, [c65 + jnp.full((16,), c, jnp.int32)]
                    )
                if c >= skew:
                    rows_v[b][c - skew, pl.ds(g * 16, 16)] = vals.pop(c - skew)

    for b in range(_NBUF):
        @pl.when(b < n_my)
        def _(b=b):
            start_in(b, jnp.int32(b))

    n_outer = (_NCHUNKS // _NW + 1 + 2 * _NBUF - 1) // _NBUF

    def outer(o, carry):
        for b in range(_NBUF):
            j = o * _NBUF + b

            @pl.when(jnp.logical_and(j >= _NBUF, j - _NBUF < n_my))
            def _():
                pltpu.make_async_copy(
                    rows_v[b],
                    out_hbm.at[:, pl.ds(0, _CHUNK)],
                    o_sem[b],
                ).wait()

            @pl.when(j < n_my)
            def _():
                pltpu.make_async_copy(
                    ea_hbm.at[:, pl.ds(0, _CHUNK)], ea_v[b], in_sem[b]
                ).wait()
                process(b)
                pltpu.make_async_copy(
                    rows_v[b],
                    out_hbm.at[:, pl.ds(chunk_of(j) * _CHUNK, _CHUNK)],
                    o_sem[b],
                ).start()

                @pl.when(j + _NBUF < n_my)
                def _():
                    start_in(b, j + _NBUF)

        return carry

    lax.fori_loop(0, n_outer, outer, jnp.int32(0), unroll=False)


@jax.jit
def kernel(edge_attr, W0, W1, W2):
    table = (
        W0[:, None, None, :] + W1[None, :, None, :] + W2[None, None, :, :]
    ).reshape(_NROWS, _EMB_DIM)

    scratch = (
        [pltpu.VMEM((13, _CHUNK), jnp.float32) for _ in range(_NBUF)]
        + [pltpu.VMEM((_EMB_DIM, _CHUNK), jnp.float32) for _ in range(_NBUF)]
        + [pltpu.SemaphoreType.DMA for _ in range(2 * _NBUF)]
    )
    run = pl.kernel(
        _body,
        out_type=jax.ShapeDtypeStruct((_EMB_DIM, _E), jnp.float32),
        mesh=plsc.VectorSubcoreMesh(core_axis_name="c", subcore_axis_name="s"),
        scratch_types=[
            pltpu.VMEM((_NROWS, _EMB_DIM), jnp.float32),
            pltpu.VMEM((16 * _REP, ), jnp.float32),
        ] + scratch,
        compiler_params=pltpu.CompilerParams(
            needs_layout_passes=False, use_tc_tiling_on_sc=True
        ),
    )
    out_t = run(edge_attr.T, table)
    return out_t.T


# speedup vs baseline: 5.1449x; 1.4055x over previous
import jax
import jax.numpy as jnp
from jax import lax
from jax.experimental import pallas as pl
from jax.experimental.pallas import tpu as pltpu
from jax.experimental.pallas import tpu_sc as plsc

_SEG_DIMS = [5, 6, 2]
_EMB_DIM = 64
_E = 800000

_NC = 2
_NS = 16
_NW = _NC * _NS
_CHUNK = 128
_NCHUNKS = _E // _CHUNK
_GROUPS = _CHUNK // 16
_NBUF = 3
_NROWS = _SEG_DIMS[0] * _SEG_DIMS[1] * _SEG_DIMS[2]
_REP = _NROWS * _EMB_DIM + 1


def _seg_argmax(cols):
    best = cols[0]
    bidx = jnp.zeros((16,), jnp.int32)
    for j in range(1, len(cols)):
        m = cols[j] > best
        bidx = jnp.where(m, jnp.full((16,), j, jnp.int32), bidx)
        best = jnp.where(m, cols[j], best)
    return bidx


def _body(ea_hbm, t_hbm, out_hbm, t_tmp, t65, *scratch):
    ea_v = scratch[0:_NBUF]
    rows_v = scratch[_NBUF:2 * _NBUF]
    in_sem = scratch[2 * _NBUF:3 * _NBUF]
    o_sem = scratch[3 * _NBUF:4 * _NBUF]

    wid = lax.axis_index("s") * _NC + lax.axis_index("c")
    n_my = (_NCHUNKS - wid + _NW - 1) // _NW

    pltpu.sync_copy(t_hbm, t_tmp)

    def t_row(r, carry):
        for q in range(4):
            vals = t_tmp[r, pl.ds(q * 16, 16)]
            for rep in range(16):
                t65[pl.ds(rep * _REP + r * _EMB_DIM + q * 16, 16)] = vals
        return carry

    lax.fori_loop(0, _NROWS, t_row, jnp.int32(0))

    def chunk_of(j):
        return wid + j * _NW

    def start_in(b, j):
        base = chunk_of(j) * _CHUNK
        pltpu.make_async_copy(
            ea_hbm.at[:, pl.ds(base, _CHUNK)], ea_v[b], in_sem[b]
        ).start()

    lane_rep = lax.iota(jnp.int32, 16) * _REP

    def process(b):
        @plsc.parallel_loop(0, _GROUPS, 1, unroll=2)
        def grp(g):
            cols = [ea_v[b][j, pl.ds(g * 16, 16)] for j in range(13)]
            i0 = _seg_argmax(cols[0:5])
            i1 = _seg_argmax(cols[5:11])
            i2 = _seg_argmax(cols[11:13])
            c65 = (i0 * 12 + i1 * 2 + i2) * _EMB_DIM + lane_rep
            skew = 6
            vals = {}
            for c in range(_EMB_DIM + skew):
                if c < _EMB_DIM:
                    vals[c] = plsc.load_gather(
                        t65, [c65 + jnp.full((16,), c, jnp.int32)]
                    )
                if c >= skew:
                    rows_v[b][c - skew, pl.ds(g * 16, 16)] = vals.pop(c - skew)

    for b in range(_NBUF):
        @pl.when(b < n_my)
        def _(b=b):
            start_in(b, jnp.int32(b))

    n_outer = (_NCHUNKS // _NW + 1 + 2 * _NBUF - 1) // _NBUF

    def outer(o, carry):
        for b in range(_NBUF):
            j = o * _NBUF + b

            @pl.when(jnp.logical_and(j >= _NBUF, j - _NBUF < n_my))
            def _():
                pltpu.make_async_copy(
                    rows_v[b],
                    out_hbm.at[:, pl.ds(0, _CHUNK)],
                    o_sem[b],
                ).wait()

            @pl.when(j < n_my)
            def _():
                pltpu.make_async_copy(
                    ea_hbm.at[:, pl.ds(0, _CHUNK)], ea_v[b], in_sem[b]
                ).wait()
                process(b)
                pltpu.make_async_copy(
                    rows_v[b],
                    out_hbm.at[:, pl.ds(chunk_of(j) * _CHUNK, _CHUNK)],
                    o_sem[b],
                ).start()

                @pl.when(j + _NBUF < n_my)
                def _():
                    start_in(b, j + _NBUF)

        return carry

    lax.fori_loop(0, n_outer, outer, jnp.int32(0), unroll=False)


@jax.jit
def kernel(edge_attr, W0, W1, W2):
    table = (
        W0[:, None, None, :] + W1[None, :, None, :] + W2[None, None, :, :]
    ).reshape(_NROWS, _EMB_DIM)

    scratch = (
        [pltpu.VMEM((13, _CHUNK), jnp.float32) for _ in range(_NBUF)]
        + [pltpu.VMEM((_EMB_DIM, _CHUNK), jnp.float32) for _ in range(_NBUF)]
        + [pltpu.SemaphoreType.DMA for _ in range(2 * _NBUF)]
    )
    run = pl.kernel(
        _body,
        out_type=jax.ShapeDtypeStruct((_EMB_DIM, _E), jnp.float32),
        mesh=plsc.VectorSubcoreMesh(core_axis_name="c", subcore_axis_name="s"),
        scratch_types=[
            pltpu.VMEM((_NROWS, _EMB_DIM), jnp.float32),
            pltpu.VMEM((16 * _REP, ), jnp.float32),
        ] + scratch,
        compiler_params=pltpu.CompilerParams(
            needs_layout_passes=False, use_tc_tiling_on_sc=True
        ),
    )
    out_t = run(edge_attr.T, table)
    return out_t.T
